# baseline (speedup 1.0000x reference)
import jax
import jax.numpy as jnp
from jax.experimental import pallas as pl
from jax.experimental.pallas import tpu as pltpu

B, S, D = 4, 2048, 2048
NUM_ROUTED, TOP_K, RANK = 8, 2, 64
N_TOK = B * S

TM = 512
W1_COLS = D + 128 + 64 + NUM_ROUTED * RANK + 64
H_OFF = D + 128
H_COLS = 64 + NUM_ROUTED * RANK + 64


def _fused_kernel(x_ref, w1_ref, w2_ref, exp_ref, b_ref, br_ref, o_ref):
    xb = x_ref[...].astype(jnp.bfloat16)
    h = jax.lax.dot_general(
        xb, w1_ref[...], (((1,), (1,)), ((), ())),
        preferred_element_type=jnp.float32)

    logits = h[:, D:D + NUM_ROUTED] + br_ref[...]
    m = jnp.max(logits, axis=1, keepdims=True)
    e = jnp.exp(logits - m)
    p = e / jnp.sum(e, axis=1, keepdims=True)

    idx = jax.lax.broadcasted_iota(jnp.int32, p.shape, 1)
    m1 = jnp.max(p, axis=1, keepdims=True)
    i1 = jnp.min(jnp.where(p == m1, idx, NUM_ROUTED), axis=1, keepdims=True)
    mask1 = idx == i1
    p2 = jnp.where(mask1, -1.0, p)
    m2 = jnp.max(p2, axis=1, keepdims=True)
    i2 = jnp.min(jnp.where(p2 == m2, idx, NUM_ROUTED), axis=1, keepdims=True)
    s = jnp.where(mask1 | (idx == i2), p, 0.0)

    scale = jax.lax.dot_general(
        s.astype(jnp.bfloat16), exp_ref[...], (((1,), (0,)), ((), ())),
        preferred_element_type=jnp.float32)
    cols = jax.lax.broadcasted_iota(jnp.int32, (TM, H_COLS), 1)
    scale = jnp.where(cols < RANK, 1.0, scale)

    hs = (h[:, H_OFF:H_OFF + H_COLS] * scale).astype(jnp.bfloat16)
    out = h[:, :D] + jax.lax.dot_general(
        hs, w2_ref[...], (((1,), (1,)), ((), ())),
        preferred_element_type=jnp.float32)
    o_ref[...] = out + b_ref[...]


def _prep1_kernel(wo_ref, wr_ref, ws1_ref, wr1_ref, o_ref):
    i = pl.program_id(0)

    @pl.when(i < 8)
    def _():
        o_ref[...] = wo_ref[...].astype(jnp.bfloat16)

    @pl.when(i == 8)
    def _():
        o_ref[...] = jnp.concatenate([
            wr_ref[...],
            jnp.zeros((120, D), jnp.float32),
            ws1_ref[...],
            wr1_ref[0:64],
        ], axis=0).astype(jnp.bfloat16)

    @pl.when(i == 9)
    def _():
        o_ref[...] = wr1_ref[64:320].astype(jnp.bfloat16)

    @pl.when(i == 10)
    def _():
        o_ref[...] = jnp.concatenate([
            wr1_ref[320:512],
            jnp.zeros((64, D), jnp.float32),
        ], axis=0).astype(jnp.bfloat16)


def _prep2_kernel(ws2_ref, wr2_ref, o_ref):
    pieces = [ws2_ref[...]] + [wr2_ref[e] for e in range(NUM_ROUTED)]
    pieces.append(jnp.zeros((512, RANK), jnp.float32))
    o_ref[...] = jnp.concatenate(pieces, axis=1).astype(jnp.bfloat16)


@jax.jit
def kernel(x, W_orig, b_orig, Ws1, Ws2, Wr1, Wr2, W_router, b_router):
    xf = x.reshape(N_TOK, D)

    w1 = pl.pallas_call(
        _prep1_kernel,
        grid=(11,),
        in_specs=[
            pl.BlockSpec((256, D), lambda i: (jnp.minimum(i, 7), 0)),
            pl.BlockSpec((NUM_ROUTED, D), lambda i: (0, 0)),
            pl.BlockSpec((RANK, D), lambda i: (0, 0)),
            pl.BlockSpec((NUM_ROUTED * RANK, D), lambda i: (0, 0)),
        ],
        out_specs=pl.BlockSpec((256, D), lambda i: (i, 0)),
        out_shape=jax.ShapeDtypeStruct((W1_COLS, D), jnp.bfloat16),
        compiler_params=pltpu.CompilerParams(
            dimension_semantics=("arbitrary",),
        ),
    )(W_orig, W_router, Ws1[0], Wr1.reshape(NUM_ROUTED * RANK, D))

    w2 = pl.pallas_call(
        _prep2_kernel,
        grid=(4,),
        in_specs=[
            pl.BlockSpec((512, RANK), lambda i: (i, 0)),
            pl.BlockSpec((NUM_ROUTED, 512, RANK), lambda i: (0, i, 0)),
        ],
        out_specs=pl.BlockSpec((512, H_COLS), lambda i: (i, 0)),
        out_shape=jax.ShapeDtypeStruct((D, H_COLS), jnp.bfloat16),
        compiler_params=pltpu.CompilerParams(
            dimension_semantics=("arbitrary",),
        ),
    )(Ws2[0], Wr2)

    rows = jnp.arange(NUM_ROUTED)[:, None]
    cc = jnp.arange(H_COLS)[None, :]
    expand = ((cc >= RANK + rows * RANK) & (cc < RANK + (rows + 1) * RANK)
              ).astype(jnp.bfloat16)

    bias = b_orig[None, :]
    rbias = b_router[None, :]

    out = pl.pallas_call(
        _fused_kernel,
        grid=(N_TOK // TM,),
        in_specs=[
            pl.BlockSpec((TM, D), lambda i: (i, 0)),
            pl.BlockSpec((W1_COLS, D), lambda i: (0, 0)),
            pl.BlockSpec((D, H_COLS), lambda i: (0, 0)),
            pl.BlockSpec((NUM_ROUTED, H_COLS), lambda i: (0, 0)),
            pl.BlockSpec((1, D), lambda i: (0, 0)),
            pl.BlockSpec((1, NUM_ROUTED), lambda i: (0, 0)),
        ],
        out_specs=pl.BlockSpec((TM, D), lambda i: (i, 0)),
        out_shape=jax.ShapeDtypeStruct((N_TOK, D), jnp.float32),
        compiler_params=pltpu.CompilerParams(
            dimension_semantics=("arbitrary",),
        ),
    )(xf, w1, w2, expand, bias, rbias)

    return out.reshape(B, S, D)


# speedup vs baseline: 1.0044x; 1.0044x over previous
import jax
import jax.numpy as jnp
from jax.experimental import pallas as pl
from jax.experimental.pallas import tpu as pltpu

B, S, D = 4, 2048, 2048
NUM_ROUTED, TOP_K, RANK = 8, 2, 64
N_TOK = B * S

TM = 512
NB = N_TOK // TM
H_COLS = RANK + NUM_ROUTED * RANK + 64
H_OFF = 128
BASE_OFF = H_OFF + H_COLS
W1_ROWS = BASE_OFF + D


def _fused_kernel(x_ref, w1_ref, w2_ref, exp_ref, b_ref, br_ref, o_ref):
    xb = x_ref[...].astype(jnp.bfloat16)
    h = jax.lax.dot_general(
        xb, w1_ref[...], (((1,), (1,)), ((), ())),
        preferred_element_type=jnp.float32)

    logits = h[:, 0:NUM_ROUTED] + br_ref[...]
    m = jnp.max(logits, axis=1, keepdims=True)
    e = jnp.exp(logits - m)
    p = e / jnp.sum(e, axis=1, keepdims=True)

    idx = jax.lax.broadcasted_iota(jnp.int32, p.shape, 1)
    m1 = jnp.max(p, axis=1, keepdims=True)
    i1 = jnp.min(jnp.where(p == m1, idx, NUM_ROUTED), axis=1, keepdims=True)
    mask1 = idx == i1
    p2 = jnp.where(mask1, -1.0, p)
    m2 = jnp.max(p2, axis=1, keepdims=True)
    i2 = jnp.min(jnp.where(p2 == m2, idx, NUM_ROUTED), axis=1, keepdims=True)
    s = jnp.where(mask1 | (idx == i2), p, 0.0)

    scale = jax.lax.dot_general(
        s.astype(jnp.bfloat16), exp_ref[...], (((1,), (0,)), ((), ())),
        preferred_element_type=jnp.float32)
    cols = jax.lax.broadcasted_iota(jnp.int32, (TM, H_COLS), 1)
    scale = jnp.where(cols < RANK, 1.0, scale)

    hs = (h[:, H_OFF:H_OFF + H_COLS] * scale).astype(jnp.bfloat16)
    out = h[:, BASE_OFF:BASE_OFF + D] + jax.lax.dot_general(
        hs, w2_ref[...], (((1,), (1,)), ((), ())),
        preferred_element_type=jnp.float32)
    o_ref[...] = out + b_ref[...]


def _prep_kernel(wo_ref, wr_ref, ws1_ref, wr1_ref, ws2_ref, wr2_ref,
                 o1_ref, o2_ref):
    i = pl.program_id(0)

    @pl.when(i == 0)
    def _():
        o1_ref[...] = jnp.concatenate(
            [wr_ref[...], jnp.zeros((120, D), jnp.float32)],
            axis=0).astype(jnp.bfloat16)

    @pl.when(i == 1)
    def _():
        o1_ref[...] = jnp.concatenate(
            [ws1_ref[...], wr1_ref[0:64]], axis=0).astype(jnp.bfloat16)

    @pl.when(i == 2)
    def _():
        o1_ref[...] = wr1_ref[64:192].astype(jnp.bfloat16)

    @pl.when(i == 3)
    def _():
        o1_ref[...] = wr1_ref[192:320].astype(jnp.bfloat16)

    @pl.when(i == 4)
    def _():
        o1_ref[...] = wr1_ref[320:448].astype(jnp.bfloat16)

    @pl.when(i == 5)
    def _():
        o1_ref[...] = jnp.concatenate(
            [wr1_ref[448:512], jnp.zeros((64, D), jnp.float32)],
            axis=0).astype(jnp.bfloat16)

    @pl.when(i >= 6)
    def _():
        o1_ref[...] = wo_ref[...].astype(jnp.bfloat16)

    @pl.when(i < 4)
    def _():
        pieces = [ws2_ref[...]] + [wr2_ref[e] for e in range(NUM_ROUTED)]
        pieces.append(jnp.zeros((512, RANK), jnp.float32))
        o2_ref[...] = jnp.concatenate(pieces, axis=1).astype(jnp.bfloat16)


@jax.jit
def kernel(x, W_orig, b_orig, Ws1, Ws2, Wr1, Wr2, W_router, b_router):
    xf = x.reshape(N_TOK, D)

    w1, w2 = pl.pallas_call(
        _prep_kernel,
        grid=(22,),
        in_specs=[
            pl.BlockSpec((128, D), lambda i: (jnp.clip(i - 6, 0, 15), 0)),
            pl.BlockSpec((NUM_ROUTED, D), lambda i: (0, 0)),
            pl.BlockSpec((RANK, D), lambda i: (0, 0)),
            pl.BlockSpec((NUM_ROUTED * RANK, D), lambda i: (0, 0)),
            pl.BlockSpec((512, RANK), lambda i: (jnp.minimum(i, 3), 0)),
            pl.BlockSpec((NUM_ROUTED, 512, RANK),
                         lambda i: (0, jnp.minimum(i, 3), 0)),
        ],
        out_specs=[
            pl.BlockSpec((128, D), lambda i: (i, 0)),
            pl.BlockSpec((512, H_COLS), lambda i: (jnp.minimum(i, 3), 0)),
        ],
        out_shape=[
            jax.ShapeDtypeStruct((W1_ROWS, D), jnp.bfloat16),
            jax.ShapeDtypeStruct((D, H_COLS), jnp.bfloat16),
        ],
        compiler_params=pltpu.CompilerParams(
            dimension_semantics=("arbitrary",),
        ),
    )(W_orig, W_router, Ws1[0], Wr1.reshape(NUM_ROUTED * RANK, D),
      Ws2[0], Wr2)

    rows = jnp.arange(NUM_ROUTED)[:, None]
    cc = jnp.arange(H_COLS)[None, :]
    expand = ((cc >= RANK + rows * RANK) & (cc < RANK + (rows + 1) * RANK)
              ).astype(jnp.bfloat16)

    bias = b_orig[None, :]
    rbias = b_router[None, :]

    out = pl.pallas_call(
        _fused_kernel,
        grid=(NB,),
        in_specs=[
            pl.BlockSpec((TM, D), lambda i: (i, 0)),
            pl.BlockSpec((W1_ROWS, D), lambda i: (0, 0)),
            pl.BlockSpec((D, H_COLS), lambda i: (0, 0)),
            pl.BlockSpec((NUM_ROUTED, H_COLS), lambda i: (0, 0)),
            pl.BlockSpec((1, D), lambda i: (0, 0)),
            pl.BlockSpec((1, NUM_ROUTED), lambda i: (0, 0)),
        ],
        out_specs=pl.BlockSpec((TM, D), lambda i: (i, 0)),
        out_shape=jax.ShapeDtypeStruct((N_TOK, D), jnp.float32),
        compiler_params=pltpu.CompilerParams(
            dimension_semantics=("arbitrary",),
        ),
    )(xf, w1, w2, expand, bias, rbias)

    return out.reshape(B, S, D)
